# Initial kernel scaffold; baseline (speedup 1.0000x reference)
import jax
import jax.numpy as jnp
from jax.experimental import pallas as pl


def kernel(x, edge_index, W1, b1, W2, b2, Wl, bl):
    raise NotImplementedError("write your pallas kernel here")


# speedup vs baseline: 13.0059x; 13.0059x over previous
import functools

import jax
import jax.numpy as jnp
from jax import lax
from jax.experimental import pallas as pl
from jax.experimental.pallas import tpu as pltpu
from jax.experimental.pallas import tpu_sc as plsc

_CHUNK = 80


def _sc_info():
    try:
        info = plsc.get_sparse_core_info()
        return info.num_cores, info.num_subcores
    except Exception:
        return 2, 16


_ZCHUNK = 2000


@functools.lru_cache(maxsize=None)
def _make_deg_kernel(N_acc, E_pad, nc, ns):
    nw = nc * ns
    ept = E_pad // nw
    nchunk = ept // _CHUNK
    mesh = plsc.VectorSubcoreMesh(core_axis_name="c", subcore_axis_name="s")

    @functools.partial(
        pl.kernel,
        mesh=mesh,
        out_type=[jax.ShapeDtypeStruct((N_acc,), jnp.float32),
                  jax.ShapeDtypeStruct((N_acc,), jnp.float32)],
        scratch_types=[
            pltpu.VMEM((_CHUNK,), jnp.int32),
            pltpu.VMEM((_CHUNK,), jnp.float32),
            pltpu.VMEM((_ZCHUNK,), jnp.float32),
            pltpu.VMEM_SHARED((N_acc,), jnp.float32),
        ],
    )
    def deg_kernel(dst_hbm, out0_hbm, out1_hbm, dst_v, ones_v, zero_v, acc_sh):
        c = lax.axis_index("c")
        s = lax.axis_index("s")
        wid = s * nc + c

        for k in range(_CHUNK // 16):
            ones_v[pl.ds(k * 16, 16)] = jnp.full((16,), 1.0, jnp.float32)

        @pl.when(s == 0)
        def _init():
            for k in range(_ZCHUNK // 16):
                zero_v[pl.ds(k * 16, 16)] = jnp.zeros((16,), jnp.float32)
            for k in range(N_acc // _ZCHUNK):
                pltpu.sync_copy(zero_v, acc_sh.at[pl.ds(k * _ZCHUNK, _ZCHUNK)])
            rem = N_acc % _ZCHUNK
            if rem:
                pltpu.sync_copy(zero_v.at[pl.ds(0, rem)],
                                acc_sh.at[pl.ds(N_acc - rem, rem)])

        plsc.subcore_barrier()

        def body(j, carry):
            off = wid * ept + j * _CHUNK
            pltpu.sync_copy(dst_hbm.at[pl.ds(off, _CHUNK)], dst_v)
            pltpu.sync_copy(ones_v, acc_sh.at[dst_v], add=True)
            return carry

        lax.fori_loop(0, nchunk, body, 0)
        plsc.subcore_barrier()

        @pl.when(jnp.logical_and(s == 0, c == 0))
        def _writeout0():
            pltpu.sync_copy(acc_sh, out0_hbm)

        @pl.when(jnp.logical_and(s == 0, c == 1))
        def _writeout1():
            pltpu.sync_copy(acc_sh, out1_hbm)

    return deg_kernel


@functools.lru_cache(maxsize=None)
def _make_scatter_kernel(N, N_acc, D, E_pad, nc, ns):
    nw = nc * ns
    ept = E_pad // nw
    nchunk = ept // _CHUNK
    mesh = plsc.VectorSubcoreMesh(core_axis_name="c", subcore_axis_name="s")

    @functools.partial(
        pl.kernel,
        mesh=mesh,
        out_type=jax.ShapeDtypeStruct((nc, N_acc, D), jnp.float32),
        scratch_types=[
            pltpu.VMEM((_CHUNK,), jnp.int32),
            pltpu.VMEM((_CHUNK,), jnp.int32),
            pltpu.VMEM((_CHUNK, D), jnp.float32),
            pltpu.VMEM_SHARED((N_acc, D), jnp.float32),
            pltpu.SemaphoreType.DMA,
        ],
    )
    def scatter_kernel(g_hbm, src_hbm, dst_hbm, zeros_hbm, out_hbm,
                       src_v, dst_v, rows_v, acc_sh, sem):
        c = lax.axis_index("c")
        s = lax.axis_index("s")
        wid = s * nc + c

        @pl.when(s == 0)
        def _init():
            pltpu.sync_copy(zeros_hbm, acc_sh)

        plsc.subcore_barrier()

        def body(j, carry):
            off = wid * ept + j * _CHUNK
            pltpu.sync_copy(src_hbm.at[pl.ds(off, _CHUNK)], src_v)
            pltpu.sync_copy(dst_hbm.at[pl.ds(off, _CHUNK)], dst_v)
            pltpu.async_copy(g_hbm.at[src_v], rows_v, sem).wait()
            pltpu.sync_copy(rows_v, acc_sh.at[dst_v], add=True)
            return carry

        lax.fori_loop(0, nchunk, body, 0)
        plsc.subcore_barrier()

        @pl.when(s == 0)
        def _writeout():
            pltpu.sync_copy(acc_sh, out_hbm.at[c])

    return scatter_kernel


def _dinv_from_parts(d0, d1):
    return lax.rsqrt(d0 + d1 + 1.0)


def _k1_body(x_ref, w1_ref, d0_ref, d1_ref, g1_ref):
    dinv = _dinv_from_parts(d0_ref[...], d1_ref[...])
    p = jnp.dot(x_ref[...], w1_ref[...], preferred_element_type=jnp.float32)
    g1_ref[...] = p * dinv


def _k2_body(s1_ref, g1_ref, d0_ref, d1_ref, b1_ref, w2_ref, g2_ref):
    dinv = _dinv_from_parts(d0_ref[...], d1_ref[...])
    h1 = jnp.maximum(dinv * (s1_ref[0] + s1_ref[1] + g1_ref[...]) + b1_ref[...], 0.0)
    g2_ref[...] = jnp.dot(h1, w2_ref[...], preferred_element_type=jnp.float32) * dinv


def _k3_body(s2_ref, g2_ref, d0_ref, d1_ref, b2_ref, wl_ref, bl_ref, out_ref):
    dinv = _dinv_from_parts(d0_ref[...], d1_ref[...])
    h2 = dinv * (s2_ref[0] + s2_ref[1] + g2_ref[...]) + b2_ref[...]
    out_ref[...] = jnp.maximum(
        jnp.dot(h2, wl_ref[...], preferred_element_type=jnp.float32) + bl_ref[...], 0.0)


def kernel(x, edge_index, W1, b1, W2, b2, Wl, bl):
    N, d_in = x.shape
    d_hid = W1.shape[1]
    d_out = W2.shape[1]
    d_emb = Wl.shape[1]
    E = edge_index.shape[1]
    nc, ns = _sc_info()
    nw = nc * ns

    src = edge_index[0]
    dst = edge_index[1]
    quantum = nw * _CHUNK
    E_pad = ((E + quantum - 1) // quantum) * quantum
    if E_pad != E:
        pad = E_pad - E
        src = jnp.concatenate([src, jnp.zeros((pad,), src.dtype)])
        dst = jnp.concatenate([dst, jnp.full((pad,), N, dst.dtype)])
        N_acc = N + 8
    else:
        N_acc = N

    zeros_d = jnp.zeros((N_acc, d_hid), jnp.float32)

    deg0, deg1 = _make_deg_kernel(N_acc, E_pad, nc, ns)(dst)
    deg0 = deg0.reshape(N_acc, 1)
    deg1 = deg1.reshape(N_acc, 1)
    scat = _make_scatter_kernel(N, N_acc, d_hid, E_pad, nc, ns)
    scat2 = (scat if d_out == d_hid
             else _make_scatter_kernel(N, N_acc, d_out, E_pad, nc, ns))

    B = 2000
    grid = (N // B,)
    deg_spec = pl.BlockSpec((B, 1), lambda i: (i, 0))
    rows_spec = pl.BlockSpec((B, d_hid), lambda i: (i, 0))
    parts_spec = pl.BlockSpec((2, B, d_hid), lambda i: (0, i, 0))

    g1 = pl.pallas_call(
        _k1_body,
        grid=grid,
        in_specs=[
            pl.BlockSpec((B, d_in), lambda i: (i, 0)),
            pl.BlockSpec((d_in, d_hid), lambda i: (0, 0)),
            deg_spec,
            deg_spec,
        ],
        out_specs=rows_spec,
        out_shape=jax.ShapeDtypeStruct((N, d_hid), jnp.float32),
    )(x, W1, deg0, deg1)

    s1 = scat(g1, src, dst, zeros_d)

    g2 = pl.pallas_call(
        _k2_body,
        grid=grid,
        in_specs=[
            parts_spec,
            rows_spec,
            deg_spec,
            deg_spec,
            pl.BlockSpec((1, d_hid), lambda i: (0, 0)),
            pl.BlockSpec((d_hid, d_out), lambda i: (0, 0)),
        ],
        out_specs=pl.BlockSpec((B, d_out), lambda i: (i, 0)),
        out_shape=jax.ShapeDtypeStruct((N, d_out), jnp.float32),
    )(s1, g1, deg0, deg1, b1.reshape(1, -1), W2)

    s2 = scat2(g2, src, dst,
               zeros_d if d_out == d_hid else jnp.zeros((N_acc, d_out), jnp.float32))

    out = pl.pallas_call(
        _k3_body,
        grid=grid,
        in_specs=[
            parts_spec,
            pl.BlockSpec((B, d_out), lambda i: (i, 0)),
            deg_spec,
            deg_spec,
            pl.BlockSpec((1, d_out), lambda i: (0, 0)),
            pl.BlockSpec((d_out, d_emb), lambda i: (0, 0)),
            pl.BlockSpec((1, d_emb), lambda i: (0, 0)),
        ],
        out_specs=pl.BlockSpec((B, d_emb), lambda i: (i, 0)),
        out_shape=jax.ShapeDtypeStruct((N, d_emb), jnp.float32),
    )(s2, g2, deg0, deg1, b2.reshape(1, -1), Wl, bl.reshape(1, -1))

    return out


# speedup vs baseline: 18.5258x; 1.4244x over previous
import functools

import jax
import jax.numpy as jnp
from jax import lax
from jax.experimental import pallas as pl
from jax.experimental.pallas import tpu as pltpu
from jax.experimental.pallas import tpu_sc as plsc

_CHUNK = 80


def _sc_info():
    try:
        info = plsc.get_sparse_core_info()
        return info.num_cores, info.num_subcores
    except Exception:
        return 2, 16


_ZCHUNK = 2000


@functools.lru_cache(maxsize=None)
def _make_deg_kernel(N_acc, E_pad, nc, ns):
    nw = nc * ns
    ept = E_pad // nw
    nchunk = ept // _CHUNK
    mesh = plsc.VectorSubcoreMesh(core_axis_name="c", subcore_axis_name="s")

    @functools.partial(
        pl.kernel,
        mesh=mesh,
        out_type=[jax.ShapeDtypeStruct((N_acc,), jnp.float32),
                  jax.ShapeDtypeStruct((N_acc,), jnp.float32)],
        scratch_types=[
            pltpu.VMEM((ept,), jnp.int32),
            pltpu.VMEM((_CHUNK,), jnp.int32),
            pltpu.VMEM((_CHUNK,), jnp.float32),
            pltpu.VMEM((_ZCHUNK,), jnp.float32),
            pltpu.VMEM_SHARED((N_acc,), jnp.float32),
        ],
    )
    def deg_kernel(dst_hbm, out0_hbm, out1_hbm, dst_all, dst_v, ones_v, zero_v,
                   acc_sh):
        c = lax.axis_index("c")
        s = lax.axis_index("s")
        wid = s * nc + c

        pltpu.sync_copy(dst_hbm.at[pl.ds(wid * ept, ept)], dst_all)

        for k in range(_CHUNK // 16):
            ones_v[pl.ds(k * 16, 16)] = jnp.full((16,), 1.0, jnp.float32)

        @pl.when(s == 0)
        def _init():
            for k in range(_ZCHUNK // 16):
                zero_v[pl.ds(k * 16, 16)] = jnp.zeros((16,), jnp.float32)
            for k in range(N_acc // _ZCHUNK):
                pltpu.sync_copy(zero_v, acc_sh.at[pl.ds(k * _ZCHUNK, _ZCHUNK)])
            rem = N_acc % _ZCHUNK
            if rem:
                pltpu.sync_copy(zero_v.at[pl.ds(0, rem)],
                                acc_sh.at[pl.ds(N_acc - rem, rem)])

        plsc.subcore_barrier()

        def body(j, carry):
            off = j * _CHUNK
            for k in range(_CHUNK // 16):
                dst_v[pl.ds(k * 16, 16)] = dst_all[pl.ds(off + k * 16, 16)]
            pltpu.sync_copy(ones_v, acc_sh.at[dst_v], add=True)
            return carry

        lax.fori_loop(0, nchunk, body, 0)
        plsc.subcore_barrier()

        @pl.when(jnp.logical_and(s == 0, c == 0))
        def _writeout0():
            pltpu.sync_copy(acc_sh, out0_hbm)

        @pl.when(jnp.logical_and(s == 0, c == 1))
        def _writeout1():
            pltpu.sync_copy(acc_sh, out1_hbm)

    return deg_kernel


@functools.lru_cache(maxsize=None)
def _make_scatter_kernel(N, N_acc, D, E_pad, nc, ns):
    nw = nc * ns
    ept = E_pad // nw
    nchunk = ept // _CHUNK
    assert nchunk % 2 == 0, "edge padding must make the per-tile chunk count even"
    n2 = nchunk // 2
    nz = N_acc // _CHUNK
    zrem = N_acc % _CHUNK
    mesh = plsc.VectorSubcoreMesh(core_axis_name="c", subcore_axis_name="s")

    @functools.partial(
        pl.kernel,
        mesh=mesh,
        out_type=jax.ShapeDtypeStruct((nc, N_acc, D), jnp.float32),
        scratch_types=[
            pltpu.VMEM((ept,), jnp.int32),
            pltpu.VMEM((ept,), jnp.int32),
            pltpu.VMEM((_CHUNK,), jnp.int32),
            pltpu.VMEM((_CHUNK,), jnp.int32),
            pltpu.VMEM((_CHUNK, D), jnp.float32),
            pltpu.VMEM((_CHUNK, D), jnp.float32),
            pltpu.VMEM_SHARED((N_acc, D), jnp.float32),
            pltpu.SemaphoreType.DMA,
            pltpu.SemaphoreType.DMA,
        ],
    )
    def scatter_kernel(g_hbm, src_hbm, dst_hbm, out_hbm,
                       src_all, dst_all, dst_v0, dst_v1, rows0, rows1,
                       acc_sh, sem0, sem1):
        c = lax.axis_index("c")
        s = lax.axis_index("s")
        wid = s * nc + c
        base = wid * ept

        pltpu.sync_copy(src_hbm.at[pl.ds(base, ept)], src_all)
        pltpu.sync_copy(dst_hbm.at[pl.ds(base, ept)], dst_all)

        def zfill(r, carry):
            for k in range(D // 16):
                rows0[r, pl.ds(k * 16, 16)] = jnp.zeros((16,), jnp.float32)
            return carry

        lax.fori_loop(0, _CHUNK, zfill, 0)

        def zero_acc(k, carry):
            @pl.when(lax.rem(k, ns) == s)
            def _():
                pltpu.sync_copy(rows0, acc_sh.at[pl.ds(k * _CHUNK, _CHUNK)])
            return carry

        lax.fori_loop(0, nz, zero_acc, 0)
        if zrem:
            @pl.when(s == 0)
            def _zero_tail():
                pltpu.sync_copy(rows0.at[pl.ds(0, zrem)],
                                acc_sh.at[pl.ds(N_acc - zrem, zrem)])
        plsc.subcore_barrier()

        def copy_dst(j, dst_v):
            off = j * _CHUNK
            for k in range(_CHUNK // 16):
                dst_v[pl.ds(k * 16, 16)] = dst_all[pl.ds(off + k * 16, 16)]

        def gather(j, rows, sem):
            idx = src_all.at[pl.ds(j * _CHUNK, _CHUNK)]
            pltpu.async_copy(g_hbm.at[idx], rows, sem)

        def gwait(rows, sem):
            idx = src_all.at[pl.ds(0, _CHUNK)]
            pltpu.make_async_copy(g_hbm.at[idx], rows, sem).wait()

        def scatter(dst_v, rows):
            pltpu.sync_copy(rows, acc_sh.at[dst_v], add=True)

        copy_dst(0, dst_v0)
        gather(0, rows0, sem0)

        def body(jj, carry):
            e = 2 * jj
            copy_dst(e + 1, dst_v1)
            gather(e + 1, rows1, sem1)
            gwait(rows0, sem0)
            scatter(dst_v0, rows0)

            @pl.when(jj < n2 - 1)
            def _next():
                copy_dst(e + 2, dst_v0)
                gather(e + 2, rows0, sem0)

            gwait(rows1, sem1)
            scatter(dst_v1, rows1)
            return carry

        lax.fori_loop(0, n2, body, 0)
        plsc.subcore_barrier()

        def writeout(k, carry):
            @pl.when(lax.rem(k, ns) == s)
            def _():
                pltpu.sync_copy(acc_sh.at[pl.ds(k * _CHUNK, _CHUNK)],
                                out_hbm.at[c, pl.ds(k * _CHUNK, _CHUNK)])
            return carry

        lax.fori_loop(0, nz, writeout, 0)
        if zrem:
            @pl.when(s == 0)
            def _write_tail():
                pltpu.sync_copy(acc_sh.at[pl.ds(N_acc - zrem, zrem)],
                                out_hbm.at[c, pl.ds(N_acc - zrem, zrem)])

    return scatter_kernel


def _dinv_from_parts(d0, d1):
    return lax.rsqrt(d0 + d1 + 1.0)


def _k1_body(x_ref, w1_ref, d0_ref, d1_ref, g1_ref):
    dinv = _dinv_from_parts(d0_ref[...], d1_ref[...])
    p = jnp.dot(x_ref[...], w1_ref[...], preferred_element_type=jnp.float32)
    g1_ref[...] = p * dinv


def _k2_body(s1_ref, g1_ref, d0_ref, d1_ref, b1_ref, w2_ref, g2_ref):
    dinv = _dinv_from_parts(d0_ref[...], d1_ref[...])
    h1 = jnp.maximum(dinv * (s1_ref[0] + s1_ref[1] + g1_ref[...]) + b1_ref[...], 0.0)
    g2_ref[...] = jnp.dot(h1, w2_ref[...], preferred_element_type=jnp.float32) * dinv


def _k3_body(s2_ref, g2_ref, d0_ref, d1_ref, b2_ref, wl_ref, bl_ref, out_ref):
    dinv = _dinv_from_parts(d0_ref[...], d1_ref[...])
    h2 = dinv * (s2_ref[0] + s2_ref[1] + g2_ref[...]) + b2_ref[...]
    out_ref[...] = jnp.maximum(
        jnp.dot(h2, wl_ref[...], preferred_element_type=jnp.float32) + bl_ref[...], 0.0)


def kernel(x, edge_index, W1, b1, W2, b2, Wl, bl):
    N, d_in = x.shape
    d_hid = W1.shape[1]
    d_out = W2.shape[1]
    d_emb = Wl.shape[1]
    E = edge_index.shape[1]
    nc, ns = _sc_info()
    nw = nc * ns

    src = edge_index[0]
    dst = edge_index[1]
    quantum = nw * _CHUNK * 2
    E_pad = ((E + quantum - 1) // quantum) * quantum
    if E_pad != E:
        pad = E_pad - E
        src = jnp.concatenate([src, jnp.zeros((pad,), src.dtype)])
        dst = jnp.concatenate([dst, jnp.full((pad,), N, dst.dtype)])
        N_acc = N + 8
    else:
        N_acc = N

    deg0, deg1 = _make_deg_kernel(N_acc, E_pad, nc, ns)(dst)
    deg0 = deg0.reshape(N_acc, 1)
    deg1 = deg1.reshape(N_acc, 1)
    scat = _make_scatter_kernel(N, N_acc, d_hid, E_pad, nc, ns)
    scat2 = (scat if d_out == d_hid
             else _make_scatter_kernel(N, N_acc, d_out, E_pad, nc, ns))

    B = 2000
    grid = (N // B,)
    deg_spec = pl.BlockSpec((B, 1), lambda i: (i, 0))
    rows_spec = pl.BlockSpec((B, d_hid), lambda i: (i, 0))
    parts_spec = pl.BlockSpec((2, B, d_hid), lambda i: (0, i, 0))

    g1 = pl.pallas_call(
        _k1_body,
        grid=grid,
        in_specs=[
            pl.BlockSpec((B, d_in), lambda i: (i, 0)),
            pl.BlockSpec((d_in, d_hid), lambda i: (0, 0)),
            deg_spec,
            deg_spec,
        ],
        out_specs=rows_spec,
        out_shape=jax.ShapeDtypeStruct((N, d_hid), jnp.float32),
    )(x, W1, deg0, deg1)

    s1 = scat(g1, src, dst)

    g2 = pl.pallas_call(
        _k2_body,
        grid=grid,
        in_specs=[
            parts_spec,
            rows_spec,
            deg_spec,
            deg_spec,
            pl.BlockSpec((1, d_hid), lambda i: (0, 0)),
            pl.BlockSpec((d_hid, d_out), lambda i: (0, 0)),
        ],
        out_specs=pl.BlockSpec((B, d_out), lambda i: (i, 0)),
        out_shape=jax.ShapeDtypeStruct((N, d_out), jnp.float32),
    )(s1, g1, deg0, deg1, b1.reshape(1, -1), W2)

    s2 = scat2(g2, src, dst)

    out = pl.pallas_call(
        _k3_body,
        grid=grid,
        in_specs=[
            parts_spec,
            pl.BlockSpec((B, d_out), lambda i: (i, 0)),
            deg_spec,
            deg_spec,
            pl.BlockSpec((1, d_out), lambda i: (0, 0)),
            pl.BlockSpec((d_out, d_emb), lambda i: (0, 0)),
            pl.BlockSpec((1, d_emb), lambda i: (0, 0)),
        ],
        out_specs=pl.BlockSpec((B, d_emb), lambda i: (i, 0)),
        out_shape=jax.ShapeDtypeStruct((N, d_emb), jnp.float32),
    )(s2, g2, deg0, deg1, b2.reshape(1, -1), Wl, bl.reshape(1, -1))

    return out


# speedup vs baseline: 30.5175x; 1.6473x over previous
import functools

import jax
import jax.numpy as jnp
from jax import lax
from jax.experimental import pallas as pl
from jax.experimental.pallas import tpu as pltpu
from jax.experimental.pallas import tpu_sc as plsc

_CHUNK = 80


def _sc_info():
    try:
        info = plsc.get_sparse_core_info()
        return info.num_cores, info.num_subcores
    except Exception:
        return 2, 16


_ZCHUNK = 2000


@functools.lru_cache(maxsize=None)
def _make_deg_kernel(N_acc, E_pad, nc, ns):
    nw = nc * ns
    ept = E_pad // nw
    nchunk = ept // _CHUNK
    mesh = plsc.VectorSubcoreMesh(core_axis_name="c", subcore_axis_name="s")

    @functools.partial(
        pl.kernel,
        mesh=mesh,
        out_type=[jax.ShapeDtypeStruct((N_acc,), jnp.float32),
                  jax.ShapeDtypeStruct((N_acc,), jnp.float32)],
        scratch_types=[
            pltpu.VMEM((ept,), jnp.int32),
            pltpu.VMEM((_CHUNK,), jnp.int32),
            pltpu.VMEM((_CHUNK,), jnp.float32),
            pltpu.VMEM((_ZCHUNK,), jnp.float32),
            pltpu.VMEM_SHARED((N_acc,), jnp.float32),
        ],
    )
    def deg_kernel(dst_hbm, out0_hbm, out1_hbm, dst_all, dst_v, ones_v, zero_v,
                   acc_sh):
        c = lax.axis_index("c")
        s = lax.axis_index("s")
        wid = s * nc + c

        pltpu.sync_copy(dst_hbm.at[pl.ds(wid * ept, ept)], dst_all)

        for k in range(_CHUNK // 16):
            ones_v[pl.ds(k * 16, 16)] = jnp.full((16,), 1.0, jnp.float32)

        @pl.when(s == 0)
        def _init():
            for k in range(_ZCHUNK // 16):
                zero_v[pl.ds(k * 16, 16)] = jnp.zeros((16,), jnp.float32)
            for k in range(N_acc // _ZCHUNK):
                pltpu.sync_copy(zero_v, acc_sh.at[pl.ds(k * _ZCHUNK, _ZCHUNK)])
            rem = N_acc % _ZCHUNK
            if rem:
                pltpu.sync_copy(zero_v.at[pl.ds(0, rem)],
                                acc_sh.at[pl.ds(N_acc - rem, rem)])

        plsc.subcore_barrier()

        def body(j, carry):
            off = j * _CHUNK
            for k in range(_CHUNK // 16):
                dst_v[pl.ds(k * 16, 16)] = dst_all[pl.ds(off + k * 16, 16)]
            pltpu.sync_copy(ones_v, acc_sh.at[dst_v], add=True)
            return carry

        lax.fori_loop(0, nchunk, body, 0)
        plsc.subcore_barrier()

        @pl.when(jnp.logical_and(s == 0, c == 0))
        def _writeout0():
            pltpu.sync_copy(acc_sh, out0_hbm)

        @pl.when(jnp.logical_and(s == 0, c == 1))
        def _writeout1():
            pltpu.sync_copy(acc_sh, out1_hbm)

    return deg_kernel


@functools.lru_cache(maxsize=None)
def _make_scatter_kernel(N, N_acc, D, E_pad, nc, ns):
    nw = nc * ns
    ept = E_pad // nw
    nchunk = ept // _CHUNK
    assert nchunk >= 6
    ngrp = nchunk // 4
    ntail = nchunk % 4
    nz = N_acc // _CHUNK
    zrem = N_acc % _CHUNK
    mesh = plsc.VectorSubcoreMesh(core_axis_name="c", subcore_axis_name="s")

    @functools.partial(
        pl.kernel,
        mesh=mesh,
        out_type=jax.ShapeDtypeStruct((nc, N_acc, D), jnp.float32),
        scratch_types=[
            [pltpu.VMEM((_CHUNK,), jnp.int32)] * 4,
            [pltpu.VMEM((_CHUNK,), jnp.int32)] * 4,
            [pltpu.VMEM((_CHUNK, D), jnp.float32)] * 4,
            pltpu.VMEM_SHARED((N_acc, D), jnp.float32),
            [pltpu.SemaphoreType.DMA] * 4,
            [pltpu.SemaphoreType.DMA] * 4,
            [pltpu.SemaphoreType.DMA] * 4,
        ],
    )
    def scatter_kernel(g_hbm, src_hbm, dst_hbm, out_hbm,
                       src_v, dst_v, rows, acc_sh, sem_is, sem_id, sem_g):
        c = lax.axis_index("c")
        s = lax.axis_index("s")
        wid = s * nc + c
        base = wid * ept

        def idx_issue(j, b):
            off = base + j * _CHUNK
            pltpu.async_copy(src_hbm.at[pl.ds(off, _CHUNK)], src_v[b], sem_is[b])
            pltpu.async_copy(dst_hbm.at[pl.ds(off, _CHUNK)], dst_v[b], sem_id[b])

        def idx_wait_src(b):
            pltpu.make_async_copy(src_hbm.at[pl.ds(0, _CHUNK)], src_v[b],
                                  sem_is[b]).wait()

        def idx_wait_dst(b):
            pltpu.make_async_copy(dst_hbm.at[pl.ds(0, _CHUNK)], dst_v[b],
                                  sem_id[b]).wait()

        def gather_issue(b):
            pltpu.async_copy(g_hbm.at[src_v[b]], rows[b], sem_g[b])

        def gather_wait(b):
            pltpu.make_async_copy(g_hbm.at[src_v[b]], rows[b], sem_g[b]).wait()

        for t in range(4):
            if t < nchunk:
                idx_issue(t, t)

        def zfill(r, carry):
            for k in range(D // 16):
                rows[0][r, pl.ds(k * 16, 16)] = jnp.zeros((16,), jnp.float32)
            return carry

        lax.fori_loop(0, _CHUNK, zfill, 0)

        def zero_acc(k, carry):
            @pl.when(lax.rem(k, ns) == s)
            def _():
                pltpu.sync_copy(rows[0], acc_sh.at[pl.ds(k * _CHUNK, _CHUNK)])
            return carry

        lax.fori_loop(0, nz, zero_acc, 0)
        if zrem:
            @pl.when(s == 0)
            def _zero_tail():
                pltpu.sync_copy(rows[0].at[pl.ds(0, zrem)],
                                acc_sh.at[pl.ds(N_acc - zrem, zrem)])
        plsc.subcore_barrier()

        for t in range(2):
            if t < nchunk:
                idx_wait_src(t)
                gather_issue(t)

        def step(j, t):
            gather_wait(t)
            idx_wait_dst(t)
            pltpu.sync_copy(rows[t], acc_sh.at[dst_v[t]], add=True)

            @pl.when(j + 4 < nchunk)
            def _prefetch_idx():
                idx_issue(j + 4, t)

            t2 = (t + 2) % 4

            @pl.when(j + 2 < nchunk)
            def _launch_gather():
                idx_wait_src(t2)
                gather_issue(t2)

        def body(jj, carry):
            for t in range(4):
                step(4 * jj + t, t)
            return carry

        lax.fori_loop(0, ngrp, body, 0)
        for t in range(ntail):
            step(4 * ngrp + t, t)
        plsc.subcore_barrier()

        def writeout(k, carry):
            @pl.when(lax.rem(k, ns) == s)
            def _():
                pltpu.sync_copy(acc_sh.at[pl.ds(k * _CHUNK, _CHUNK)],
                                out_hbm.at[c, pl.ds(k * _CHUNK, _CHUNK)])
            return carry

        lax.fori_loop(0, nz, writeout, 0)
        if zrem:
            @pl.when(s == 0)
            def _write_tail():
                pltpu.sync_copy(acc_sh.at[pl.ds(N_acc - zrem, zrem)],
                                out_hbm.at[c, pl.ds(N_acc - zrem, zrem)])

    return scatter_kernel


def _dinv_from_parts(d0, d1):
    return lax.rsqrt(d0 + d1 + 1.0)


def _k1_body(x_ref, w1_ref, d0_ref, d1_ref, g1_ref):
    dinv = _dinv_from_parts(d0_ref[...], d1_ref[...])
    p = jnp.dot(x_ref[...], w1_ref[...], preferred_element_type=jnp.float32)
    g1_ref[...] = p * dinv


def _k2_body(s1_ref, g1_ref, d0_ref, d1_ref, b1_ref, w2_ref, g2_ref):
    dinv = _dinv_from_parts(d0_ref[...], d1_ref[...])
    h1 = jnp.maximum(dinv * (s1_ref[0] + s1_ref[1] + g1_ref[...]) + b1_ref[...], 0.0)
    g2_ref[...] = jnp.dot(h1, w2_ref[...], preferred_element_type=jnp.float32) * dinv


def _k3_body(s2_ref, g2_ref, d0_ref, d1_ref, b2_ref, wl_ref, bl_ref, out_ref):
    dinv = _dinv_from_parts(d0_ref[...], d1_ref[...])
    h2 = dinv * (s2_ref[0] + s2_ref[1] + g2_ref[...]) + b2_ref[...]
    out_ref[...] = jnp.maximum(
        jnp.dot(h2, wl_ref[...], preferred_element_type=jnp.float32) + bl_ref[...], 0.0)


def kernel(x, edge_index, W1, b1, W2, b2, Wl, bl):
    N, d_in = x.shape
    d_hid = W1.shape[1]
    d_out = W2.shape[1]
    d_emb = Wl.shape[1]
    E = edge_index.shape[1]
    nc, ns = _sc_info()
    nw = nc * ns

    src = edge_index[0]
    dst = edge_index[1]
    quantum = nw * _CHUNK
    E_pad = ((E + quantum - 1) // quantum) * quantum
    if E_pad != E:
        pad = E_pad - E
        src = jnp.concatenate([src, jnp.zeros((pad,), src.dtype)])
        dst = jnp.concatenate([dst, jnp.full((pad,), N, dst.dtype)])
        N_acc = N + 8
    else:
        N_acc = N

    deg0, deg1 = _make_deg_kernel(N_acc, E_pad, nc, ns)(dst)
    deg0 = deg0.reshape(N_acc, 1)
    deg1 = deg1.reshape(N_acc, 1)
    scat = _make_scatter_kernel(N, N_acc, d_hid, E_pad, nc, ns)
    scat2 = (scat if d_out == d_hid
             else _make_scatter_kernel(N, N_acc, d_out, E_pad, nc, ns))

    B = 2000
    grid = (N // B,)
    deg_spec = pl.BlockSpec((B, 1), lambda i: (i, 0))
    rows_spec = pl.BlockSpec((B, d_hid), lambda i: (i, 0))
    parts_spec = pl.BlockSpec((2, B, d_hid), lambda i: (0, i, 0))

    g1 = pl.pallas_call(
        _k1_body,
        grid=grid,
        in_specs=[
            pl.BlockSpec((B, d_in), lambda i: (i, 0)),
            pl.BlockSpec((d_in, d_hid), lambda i: (0, 0)),
            deg_spec,
            deg_spec,
        ],
        out_specs=rows_spec,
        out_shape=jax.ShapeDtypeStruct((N, d_hid), jnp.float32),
    )(x, W1, deg0, deg1)

    s1 = scat(g1, src, dst)

    g2 = pl.pallas_call(
        _k2_body,
        grid=grid,
        in_specs=[
            parts_spec,
            rows_spec,
            deg_spec,
            deg_spec,
            pl.BlockSpec((1, d_hid), lambda i: (0, 0)),
            pl.BlockSpec((d_hid, d_out), lambda i: (0, 0)),
        ],
        out_specs=pl.BlockSpec((B, d_out), lambda i: (i, 0)),
        out_shape=jax.ShapeDtypeStruct((N, d_out), jnp.float32),
    )(s1, g1, deg0, deg1, b1.reshape(1, -1), W2)

    s2 = scat2(g2, src, dst)

    out = pl.pallas_call(
        _k3_body,
        grid=grid,
        in_specs=[
            parts_spec,
            pl.BlockSpec((B, d_out), lambda i: (i, 0)),
            deg_spec,
            deg_spec,
            pl.BlockSpec((1, d_out), lambda i: (0, 0)),
            pl.BlockSpec((d_out, d_emb), lambda i: (0, 0)),
            pl.BlockSpec((1, d_emb), lambda i: (0, 0)),
        ],
        out_specs=pl.BlockSpec((B, d_emb), lambda i: (i, 0)),
        out_shape=jax.ShapeDtypeStruct((N, d_emb), jnp.float32),
    )(s2, g2, deg0, deg1, b2.reshape(1, -1), Wl, bl.reshape(1, -1))

    return out


# speedup vs baseline: 32.2671x; 1.0573x over previous
import functools

import jax
import jax.numpy as jnp
from jax import lax
from jax.experimental import pallas as pl
from jax.experimental.pallas import tpu as pltpu
from jax.experimental.pallas import tpu_sc as plsc

_CHUNK = 80


def _sc_info():
    try:
        info = plsc.get_sparse_core_info()
        return info.num_cores, info.num_subcores
    except Exception:
        return 2, 16


_ZCHUNK = 2000


@functools.lru_cache(maxsize=None)
def _make_deg_kernel(N_acc, E_pad, nc, ns):
    nw = nc * ns
    ept = E_pad // nw
    nchunk = ept // _CHUNK
    mesh = plsc.VectorSubcoreMesh(core_axis_name="c", subcore_axis_name="s")

    @functools.partial(
        pl.kernel,
        mesh=mesh,
        out_type=[jax.ShapeDtypeStruct((N_acc,), jnp.float32),
                  jax.ShapeDtypeStruct((N_acc,), jnp.float32)],
        scratch_types=[
            pltpu.VMEM((ept,), jnp.int32),
            [pltpu.VMEM((_CHUNK,), jnp.int32)] * 2,
            pltpu.VMEM((_CHUNK,), jnp.float32),
            pltpu.VMEM((_ZCHUNK,), jnp.float32),
            pltpu.VMEM_SHARED((N_acc,), jnp.float32),
            [pltpu.SemaphoreType.DMA] * 2,
        ],
    )
    def deg_kernel(edges_hbm, out0_hbm, out1_hbm, dst_all, dst_v, ones_v,
                   zero_v, acc_sh, sems):
        c = lax.axis_index("c")
        s = lax.axis_index("s")
        wid = s * nc + c

        pltpu.sync_copy(edges_hbm.at[pl.ds(E_pad + wid * ept, ept)], dst_all)

        for k in range(_CHUNK // 16):
            ones_v[pl.ds(k * 16, 16)] = jnp.full((16,), 1.0, jnp.float32)

        @pl.when(s == 0)
        def _init():
            for k in range(_ZCHUNK // 16):
                zero_v[pl.ds(k * 16, 16)] = jnp.zeros((16,), jnp.float32)
            for k in range(N_acc // _ZCHUNK):
                pltpu.sync_copy(zero_v, acc_sh.at[pl.ds(k * _ZCHUNK, _ZCHUNK)])
            rem = N_acc % _ZCHUNK
            if rem:
                pltpu.sync_copy(zero_v.at[pl.ds(0, rem)],
                                acc_sh.at[pl.ds(N_acc - rem, rem)])

        plsc.subcore_barrier()

        def step(j, t):
            @pl.when(j >= 2)
            def _drain():
                pltpu.make_async_copy(ones_v, acc_sh.at[dst_v[t]],
                                      sems[t]).wait()
            off = j * _CHUNK
            for k in range(_CHUNK // 16):
                dst_v[t][pl.ds(k * 16, 16)] = dst_all[pl.ds(off + k * 16, 16)]
            pltpu.async_copy(ones_v, acc_sh.at[dst_v[t]], sems[t], add=True)

        def body(jj, carry):
            step(2 * jj, 0)
            step(2 * jj + 1, 1)
            return carry

        lax.fori_loop(0, nchunk // 2, body, 0)
        if nchunk % 2:
            step(nchunk - 1, (nchunk - 1) % 2)
        for j in (nchunk - 2, nchunk - 1):
            if j >= 0:
                pltpu.make_async_copy(ones_v, acc_sh.at[dst_v[j % 2]],
                                      sems[j % 2]).wait()
        plsc.subcore_barrier()

        @pl.when(jnp.logical_and(s == 0, c == 0))
        def _writeout0():
            pltpu.sync_copy(acc_sh, out0_hbm)

        @pl.when(jnp.logical_and(s == 0, c == 1))
        def _writeout1():
            pltpu.sync_copy(acc_sh, out1_hbm)

    return deg_kernel


@functools.lru_cache(maxsize=None)
def _make_scatter_kernel(N, N_acc, D, E_pad, nc, ns):
    nw = nc * ns
    ept = E_pad // nw
    nchunk = ept // _CHUNK
    assert nchunk >= 6
    ngrp = nchunk // 4
    ntail = nchunk % 4
    nz = N_acc // _CHUNK
    zrem = N_acc % _CHUNK
    mesh = plsc.VectorSubcoreMesh(core_axis_name="c", subcore_axis_name="s")

    @functools.partial(
        pl.kernel,
        mesh=mesh,
        out_type=jax.ShapeDtypeStruct((nc, N_acc, D), jnp.float32),
        scratch_types=[
            [pltpu.VMEM((_CHUNK,), jnp.int32)] * 4,
            [pltpu.VMEM((_CHUNK,), jnp.int32)] * 4,
            [pltpu.VMEM((_CHUNK, D), jnp.float32)] * 4,
            pltpu.VMEM_SHARED((N_acc, D), jnp.float32),
            [pltpu.SemaphoreType.DMA] * 4,
            [pltpu.SemaphoreType.DMA] * 4,
            [pltpu.SemaphoreType.DMA] * 4,
        ],
    )
    def scatter_kernel(g_hbm, edges_hbm, out_hbm,
                       src_v, dst_v, rows, acc_sh, sem_is, sem_id, sem_g):
        c = lax.axis_index("c")
        s = lax.axis_index("s")
        wid = s * nc + c
        base = wid * ept

        def idx_issue(j, b):
            off = base + j * _CHUNK
            pltpu.async_copy(edges_hbm.at[pl.ds(off, _CHUNK)], src_v[b],
                             sem_is[b])
            pltpu.async_copy(edges_hbm.at[pl.ds(E_pad + off, _CHUNK)], dst_v[b],
                             sem_id[b])

        def idx_wait_src(b):
            pltpu.make_async_copy(edges_hbm.at[pl.ds(0, _CHUNK)], src_v[b],
                                  sem_is[b]).wait()

        def idx_wait_dst(b):
            pltpu.make_async_copy(edges_hbm.at[pl.ds(0, _CHUNK)], dst_v[b],
                                  sem_id[b]).wait()

        def gather_issue(b):
            pltpu.async_copy(g_hbm.at[src_v[b]], rows[b], sem_g[b])

        def gather_wait(b):
            pltpu.make_async_copy(g_hbm.at[src_v[b]], rows[b], sem_g[b]).wait()

        for t in range(4):
            if t < nchunk:
                idx_issue(t, t)

        def zfill(r, carry):
            for k in range(D // 16):
                rows[0][r, pl.ds(k * 16, 16)] = jnp.zeros((16,), jnp.float32)
            return carry

        lax.fori_loop(0, _CHUNK, zfill, 0)

        def zero_acc(k, carry):
            @pl.when(lax.rem(k, ns) == s)
            def _():
                pltpu.sync_copy(rows[0], acc_sh.at[pl.ds(k * _CHUNK, _CHUNK)])
            return carry

        lax.fori_loop(0, nz, zero_acc, 0)
        if zrem:
            @pl.when(s == 0)
            def _zero_tail():
                pltpu.sync_copy(rows[0].at[pl.ds(0, zrem)],
                                acc_sh.at[pl.ds(N_acc - zrem, zrem)])
        plsc.subcore_barrier()

        for t in range(2):
            if t < nchunk:
                idx_wait_src(t)
                gather_issue(t)

        def step(j, t):
            gather_wait(t)
            idx_wait_dst(t)
            pltpu.sync_copy(rows[t], acc_sh.at[dst_v[t]], add=True)

            @pl.when(j + 4 < nchunk)
            def _prefetch_idx():
                idx_issue(j + 4, t)

            t2 = (t + 2) % 4

            @pl.when(j + 2 < nchunk)
            def _launch_gather():
                idx_wait_src(t2)
                gather_issue(t2)

        def body(jj, carry):
            for t in range(4):
                step(4 * jj + t, t)
            return carry

        lax.fori_loop(0, ngrp, body, 0)
        for t in range(ntail):
            step(4 * ngrp + t, t)
        plsc.subcore_barrier()

        def writeout(k, carry):
            @pl.when(lax.rem(k, ns) == s)
            def _():
                pltpu.sync_copy(acc_sh.at[pl.ds(k * _CHUNK, _CHUNK)],
                                out_hbm.at[c, pl.ds(k * _CHUNK, _CHUNK)])
            return carry

        lax.fori_loop(0, nz, writeout, 0)
        if zrem:
            @pl.when(s == 0)
            def _write_tail():
                pltpu.sync_copy(acc_sh.at[pl.ds(N_acc - zrem, zrem)],
                                out_hbm.at[c, pl.ds(N_acc - zrem, zrem)])

    return scatter_kernel


def _dinv_from_parts(d0, d1):
    return lax.rsqrt(d0 + d1 + 1.0)


def _k1_body(x_ref, w1_ref, d0_ref, d1_ref, g1_ref):
    dinv = _dinv_from_parts(d0_ref[...], d1_ref[...])
    p = jnp.dot(x_ref[...], w1_ref[...], preferred_element_type=jnp.float32)
    g1_ref[...] = p * dinv


def _k2_body(s1_ref, g1_ref, d0_ref, d1_ref, b1_ref, w2_ref, g2_ref):
    dinv = _dinv_from_parts(d0_ref[...], d1_ref[...])
    h1 = jnp.maximum(dinv * (s1_ref[0] + s1_ref[1] + g1_ref[...]) + b1_ref[...], 0.0)
    g2_ref[...] = jnp.dot(h1, w2_ref[...], preferred_element_type=jnp.float32) * dinv


def _k3_body(s2_ref, g2_ref, d0_ref, d1_ref, b2_ref, wl_ref, bl_ref, out_ref):
    dinv = _dinv_from_parts(d0_ref[...], d1_ref[...])
    h2 = dinv * (s2_ref[0] + s2_ref[1] + g2_ref[...]) + b2_ref[...]
    out_ref[...] = jnp.maximum(
        jnp.dot(h2, wl_ref[...], preferred_element_type=jnp.float32) + bl_ref[...], 0.0)


def kernel(x, edge_index, W1, b1, W2, b2, Wl, bl):
    N, d_in = x.shape
    d_hid = W1.shape[1]
    d_out = W2.shape[1]
    d_emb = Wl.shape[1]
    E = edge_index.shape[1]
    nc, ns = _sc_info()
    nw = nc * ns

    quantum = nw * _CHUNK
    E_pad = ((E + quantum - 1) // quantum) * quantum
    if E_pad != E:
        pad = E_pad - E
        edges = jnp.concatenate([
            edge_index[0], jnp.zeros((pad,), edge_index.dtype),
            edge_index[1], jnp.full((pad,), N, edge_index.dtype)])
        N_acc = N + 8
    else:
        edges = edge_index.reshape(2 * E)
        N_acc = N

    deg0, deg1 = _make_deg_kernel(N_acc, E_pad, nc, ns)(edges)
    deg0 = deg0.reshape(N_acc, 1)
    deg1 = deg1.reshape(N_acc, 1)
    scat = _make_scatter_kernel(N, N_acc, d_hid, E_pad, nc, ns)
    scat2 = (scat if d_out == d_hid
             else _make_scatter_kernel(N, N_acc, d_out, E_pad, nc, ns))

    B = 2000
    grid = (N // B,)
    deg_spec = pl.BlockSpec((B, 1), lambda i: (i, 0))
    rows_spec = pl.BlockSpec((B, d_hid), lambda i: (i, 0))
    parts_spec = pl.BlockSpec((2, B, d_hid), lambda i: (0, i, 0))

    g1 = pl.pallas_call(
        _k1_body,
        grid=grid,
        in_specs=[
            pl.BlockSpec((B, d_in), lambda i: (i, 0)),
            pl.BlockSpec((d_in, d_hid), lambda i: (0, 0)),
            deg_spec,
            deg_spec,
        ],
        out_specs=rows_spec,
        out_shape=jax.ShapeDtypeStruct((N, d_hid), jnp.float32),
    )(x, W1, deg0, deg1)

    s1 = scat(g1, edges)

    g2 = pl.pallas_call(
        _k2_body,
        grid=grid,
        in_specs=[
            parts_spec,
            rows_spec,
            deg_spec,
            deg_spec,
            pl.BlockSpec((1, d_hid), lambda i: (0, 0)),
            pl.BlockSpec((d_hid, d_out), lambda i: (0, 0)),
        ],
        out_specs=pl.BlockSpec((B, d_out), lambda i: (i, 0)),
        out_shape=jax.ShapeDtypeStruct((N, d_out), jnp.float32),
    )(s1, g1, deg0, deg1, b1.reshape(1, -1), W2)

    s2 = scat2(g2, edges)

    out = pl.pallas_call(
        _k3_body,
        grid=grid,
        in_specs=[
            parts_spec,
            pl.BlockSpec((B, d_out), lambda i: (i, 0)),
            deg_spec,
            deg_spec,
            pl.BlockSpec((1, d_out), lambda i: (0, 0)),
            pl.BlockSpec((d_out, d_emb), lambda i: (0, 0)),
            pl.BlockSpec((1, d_emb), lambda i: (0, 0)),
        ],
        out_specs=pl.BlockSpec((B, d_emb), lambda i: (i, 0)),
        out_shape=jax.ShapeDtypeStruct((N, d_emb), jnp.float32),
    )(s2, g2, deg0, deg1, b2.reshape(1, -1), Wl, bl.reshape(1, -1))

    return out


# speedup vs baseline: 33.9631x; 1.0526x over previous
import functools

import jax
import jax.numpy as jnp
from jax import lax
from jax.experimental import pallas as pl
from jax.experimental.pallas import tpu as pltpu
from jax.experimental.pallas import tpu_sc as plsc

_CHUNK = 80


def _sc_info():
    try:
        info = plsc.get_sparse_core_info()
        return info.num_cores, info.num_subcores
    except Exception:
        return 2, 16


_ZCHUNK = 2000


@functools.lru_cache(maxsize=None)
def _make_deg_kernel(N_acc, E_pad, nc, ns):
    nw = nc * ns
    ept = E_pad // nw
    nchunk = ept // _CHUNK
    mesh = plsc.VectorSubcoreMesh(core_axis_name="c", subcore_axis_name="s")

    @functools.partial(
        pl.kernel,
        mesh=mesh,
        out_type=[jax.ShapeDtypeStruct((N_acc,), jnp.float32),
                  jax.ShapeDtypeStruct((N_acc,), jnp.float32)],
        scratch_types=[
            pltpu.VMEM((ept,), jnp.int32),
            [pltpu.VMEM((_CHUNK,), jnp.int32)] * 2,
            pltpu.VMEM((_CHUNK,), jnp.float32),
            pltpu.VMEM((_ZCHUNK,), jnp.float32),
            pltpu.VMEM_SHARED((N_acc,), jnp.float32),
            [pltpu.SemaphoreType.DMA] * 2,
        ],
    )
    def deg_kernel(edges_hbm, out0_hbm, out1_hbm, dst_all, dst_v, ones_v,
                   zero_v, acc_sh, sems):
        c = lax.axis_index("c")
        s = lax.axis_index("s")
        wid = s * nc + c

        pltpu.sync_copy(edges_hbm.at[pl.ds(E_pad + wid * ept, ept)], dst_all)

        for k in range(_CHUNK // 16):
            ones_v[pl.ds(k * 16, 16)] = jnp.full((16,), 1.0, jnp.float32)

        @pl.when(s == 0)
        def _init():
            for k in range(_ZCHUNK // 16):
                zero_v[pl.ds(k * 16, 16)] = jnp.zeros((16,), jnp.float32)
            for k in range(N_acc // _ZCHUNK):
                pltpu.sync_copy(zero_v, acc_sh.at[pl.ds(k * _ZCHUNK, _ZCHUNK)])
            rem = N_acc % _ZCHUNK
            if rem:
                pltpu.sync_copy(zero_v.at[pl.ds(0, rem)],
                                acc_sh.at[pl.ds(N_acc - rem, rem)])

        plsc.subcore_barrier()

        def step(j, t):
            @pl.when(j >= 2)
            def _drain():
                pltpu.make_async_copy(ones_v, acc_sh.at[dst_v[t]],
                                      sems[t]).wait()
            off = j * _CHUNK
            for k in range(_CHUNK // 16):
                dst_v[t][pl.ds(k * 16, 16)] = dst_all[pl.ds(off + k * 16, 16)]
            pltpu.async_copy(ones_v, acc_sh.at[dst_v[t]], sems[t], add=True)

        def body(jj, carry):
            step(2 * jj, 0)
            step(2 * jj + 1, 1)
            return carry

        lax.fori_loop(0, nchunk // 2, body, 0)
        if nchunk % 2:
            step(nchunk - 1, (nchunk - 1) % 2)
        for j in (nchunk - 2, nchunk - 1):
            if j >= 0:
                pltpu.make_async_copy(ones_v, acc_sh.at[dst_v[j % 2]],
                                      sems[j % 2]).wait()
        plsc.subcore_barrier()

        @pl.when(jnp.logical_and(s == 0, c == 0))
        def _writeout0():
            pltpu.sync_copy(acc_sh, out0_hbm)

        @pl.when(jnp.logical_and(s == 0, c == 1))
        def _writeout1():
            pltpu.sync_copy(acc_sh, out1_hbm)

    return deg_kernel


@functools.lru_cache(maxsize=None)
def _make_scatter_kernel(N, N_acc, D, E_pad, nc, ns):
    nw = nc * ns
    ept = E_pad // nw
    nchunk = ept // _CHUNK
    assert nchunk >= 6
    ngrp = nchunk // 4
    ntail = nchunk % 4
    nz = N_acc // _CHUNK
    zrem = N_acc % _CHUNK
    mesh = plsc.VectorSubcoreMesh(core_axis_name="c", subcore_axis_name="s")

    @functools.partial(
        pl.kernel,
        mesh=mesh,
        out_type=jax.ShapeDtypeStruct((nc, N_acc, D), jnp.float32),
        scratch_types=[
            [pltpu.VMEM((_CHUNK,), jnp.int32)] * 4,
            [pltpu.VMEM((_CHUNK,), jnp.int32)] * 4,
            [pltpu.VMEM((_CHUNK, D), jnp.float32)] * 4,
            pltpu.VMEM_SHARED((N_acc, D), jnp.float32),
            [pltpu.SemaphoreType.DMA] * 4,
            [pltpu.SemaphoreType.DMA] * 4,
            [pltpu.SemaphoreType.DMA] * 4,
        ],
    )
    def scatter_kernel(g_hbm, edges_hbm, out_hbm,
                       src_v, dst_v, rows, acc_sh, sem_is, sem_id, sem_g):
        c = lax.axis_index("c")
        s = lax.axis_index("s")
        wid = s * nc + c
        base = wid * ept

        def idx_issue(j, b):
            off = base + j * _CHUNK
            pltpu.async_copy(edges_hbm.at[pl.ds(off, _CHUNK)], src_v[b],
                             sem_is[b])
            pltpu.async_copy(edges_hbm.at[pl.ds(E_pad + off, _CHUNK)], dst_v[b],
                             sem_id[b])

        def idx_wait_src(b):
            pltpu.make_async_copy(edges_hbm.at[pl.ds(0, _CHUNK)], src_v[b],
                                  sem_is[b]).wait()

        def idx_wait_dst(b):
            pltpu.make_async_copy(edges_hbm.at[pl.ds(0, _CHUNK)], dst_v[b],
                                  sem_id[b]).wait()

        def gather_issue(b):
            pltpu.async_copy(g_hbm.at[src_v[b]], rows[b], sem_g[b])

        def gather_wait(b):
            pltpu.make_async_copy(g_hbm.at[src_v[b]], rows[b], sem_g[b]).wait()

        for t in range(4):
            if t < nchunk:
                idx_issue(t, t)

        def zfill(r, carry):
            for k in range(D // 16):
                rows[0][r, pl.ds(k * 16, 16)] = jnp.zeros((16,), jnp.float32)
            return carry

        lax.fori_loop(0, _CHUNK, zfill, 0)

        def zero_acc(k, carry):
            @pl.when(lax.rem(k, ns) == s)
            def _():
                pltpu.sync_copy(rows[0], acc_sh.at[pl.ds(k * _CHUNK, _CHUNK)])
            return carry

        lax.fori_loop(0, nz, zero_acc, 0)
        if zrem:
            @pl.when(s == 0)
            def _zero_tail():
                pltpu.sync_copy(rows[0].at[pl.ds(0, zrem)],
                                acc_sh.at[pl.ds(N_acc - zrem, zrem)])
        plsc.subcore_barrier()

        for t in range(2):
            if t < nchunk:
                idx_wait_src(t)
                gather_issue(t)

        def step(j, t):
            gather_wait(t)
            idx_wait_dst(t)
            pltpu.sync_copy(rows[t], acc_sh.at[dst_v[t]], add=True)

            @pl.when(j + 4 < nchunk)
            def _prefetch_idx():
                idx_issue(j + 4, t)

            t2 = (t + 2) % 4

            @pl.when(j + 2 < nchunk)
            def _launch_gather():
                idx_wait_src(t2)
                gather_issue(t2)

        def body(jj, carry):
            for t in range(4):
                step(4 * jj + t, t)
            return carry

        lax.fori_loop(0, ngrp, body, 0)
        for t in range(ntail):
            step(4 * ngrp + t, t)
        plsc.subcore_barrier()

        def writeout(k, carry):
            @pl.when(lax.rem(k, ns) == s)
            def _():
                pltpu.sync_copy(acc_sh.at[pl.ds(k * _CHUNK, _CHUNK)],
                                out_hbm.at[c, pl.ds(k * _CHUNK, _CHUNK)])
            return carry

        lax.fori_loop(0, nz, writeout, 0)
        if zrem:
            @pl.when(s == 0)
            def _write_tail():
                pltpu.sync_copy(acc_sh.at[pl.ds(N_acc - zrem, zrem)],
                                out_hbm.at[c, pl.ds(N_acc - zrem, zrem)])

    return scatter_kernel


def _dinv_block(d0_ref, d1_ref, B):
    i = pl.program_id(0)
    deg = d0_ref[pl.ds(i * B, B)] + d1_ref[pl.ds(i * B, B)] + 1.0
    return lax.rsqrt(deg).reshape(B, 1)


def _k1_body(x_ref, w1_ref, d0_ref, d1_ref, g1_ref):
    dinv = _dinv_block(d0_ref, d1_ref, x_ref.shape[0])
    p = jnp.dot(x_ref[...], w1_ref[...], preferred_element_type=jnp.float32)
    g1_ref[...] = p * dinv


def _k2_body(s1_ref, g1_ref, d0_ref, d1_ref, b1_ref, w2_ref, g2_ref):
    dinv = _dinv_block(d0_ref, d1_ref, g1_ref.shape[0])
    h1 = jnp.maximum(dinv * (s1_ref[0] + s1_ref[1] + g1_ref[...]) + b1_ref[...], 0.0)
    g2_ref[...] = jnp.dot(h1, w2_ref[...], preferred_element_type=jnp.float32) * dinv


def _k3_body(s2_ref, g2_ref, d0_ref, d1_ref, b2_ref, wl_ref, bl_ref, out_ref):
    dinv = _dinv_block(d0_ref, d1_ref, g2_ref.shape[0])
    h2 = dinv * (s2_ref[0] + s2_ref[1] + g2_ref[...]) + b2_ref[...]
    out_ref[...] = jnp.maximum(
        jnp.dot(h2, wl_ref[...], preferred_element_type=jnp.float32) + bl_ref[...], 0.0)


def kernel(x, edge_index, W1, b1, W2, b2, Wl, bl):
    N, d_in = x.shape
    d_hid = W1.shape[1]
    d_out = W2.shape[1]
    d_emb = Wl.shape[1]
    E = edge_index.shape[1]
    nc, ns = _sc_info()
    nw = nc * ns

    quantum = nw * _CHUNK
    E_pad = ((E + quantum - 1) // quantum) * quantum
    if E_pad != E:
        pad = E_pad - E
        edges = jnp.concatenate([
            edge_index[0], jnp.zeros((pad,), edge_index.dtype),
            edge_index[1], jnp.full((pad,), N, edge_index.dtype)])
    else:
        edges = edge_index.reshape(2 * E)

    B = 2048
    N_acc = ((N + B - 1) // B) * B
    grid = (N_acc // B,)

    deg0, deg1 = _make_deg_kernel(N_acc, E_pad, nc, ns)(edges)
    scat = _make_scatter_kernel(N, N_acc, d_hid, E_pad, nc, ns)
    scat2 = (scat if d_out == d_hid
             else _make_scatter_kernel(N, N_acc, d_out, E_pad, nc, ns))

    deg_spec = pl.BlockSpec((N_acc,), lambda i: (0,))
    rows_spec = pl.BlockSpec((B, d_hid), lambda i: (i, 0))
    parts_spec = pl.BlockSpec((2, B, d_hid), lambda i: (0, i, 0))

    g1 = pl.pallas_call(
        _k1_body,
        grid=grid,
        in_specs=[
            pl.BlockSpec((B, d_in), lambda i: (i, 0)),
            pl.BlockSpec((d_in, d_hid), lambda i: (0, 0)),
            deg_spec,
            deg_spec,
        ],
        out_specs=rows_spec,
        out_shape=jax.ShapeDtypeStruct((N_acc, d_hid), jnp.float32),
    )(x, W1, deg0, deg1)

    s1 = scat(g1, edges)

    g2 = pl.pallas_call(
        _k2_body,
        grid=grid,
        in_specs=[
            parts_spec,
            rows_spec,
            deg_spec,
            deg_spec,
            pl.BlockSpec((1, d_hid), lambda i: (0, 0)),
            pl.BlockSpec((d_hid, d_out), lambda i: (0, 0)),
        ],
        out_specs=pl.BlockSpec((B, d_out), lambda i: (i, 0)),
        out_shape=jax.ShapeDtypeStruct((N_acc, d_out), jnp.float32),
    )(s1, g1, deg0, deg1, b1.reshape(1, -1), W2)

    s2 = scat2(g2, edges)

    out = pl.pallas_call(
        _k3_body,
        grid=grid,
        in_specs=[
            parts_spec,
            pl.BlockSpec((B, d_out), lambda i: (i, 0)),
            deg_spec,
            deg_spec,
            pl.BlockSpec((1, d_out), lambda i: (0, 0)),
            pl.BlockSpec((d_out, d_emb), lambda i: (0, 0)),
            pl.BlockSpec((1, d_emb), lambda i: (0, 0)),
        ],
        out_specs=pl.BlockSpec((B, d_emb), lambda i: (i, 0)),
        out_shape=jax.ShapeDtypeStruct((N_acc, d_emb), jnp.float32),
    )(s2, g2, deg0, deg1, b2.reshape(1, -1), Wl, bl.reshape(1, -1))

    return out[:N]
